# baseline (speedup 1.0000x reference)
import functools

import jax
import jax.numpy as jnp
from jax import lax
from jax.experimental import pallas as pl
from jax.experimental.pallas import tpu as pltpu
from jax.experimental.pallas import tpu_sc as plsc

_B = 2
_D = 1024
_S = 8192
_NWD = 8
_NWS = 4
_DW = _D // _NWD
_SW = _S // _NWS
_SC = 256
_NCHUNK = _SW // _SC
_NG = _SC // 16


def _sc_body(x_hbm, pe_hbm, o_hbm, pe_v, x0_v, x1_v, sem0, sem1, sem2):
    wid = lax.axis_index("s") * 2 + lax.axis_index("c")
    d0 = (wid % _NWD) * _DW
    s_base = (wid // _NWD) * _SW
    iota = lax.iota(jnp.int32, 16)

    for chunk in range(_NCHUNK):
        s0 = s_base + chunk * _SC
        cp_pe = pltpu.async_copy(
            pe_hbm.at[pl.ds(s0, _SC), pl.ds(d0, _DW)], pe_v, sem0)
        cp_x0 = pltpu.async_copy(
            x_hbm.at[0, pl.ds(d0, _DW), pl.ds(s0, _SC)], x0_v, sem1)
        cp_x1 = pltpu.async_copy(
            x_hbm.at[1, pl.ds(d0, _DW), pl.ds(s0, _SC)], x1_v, sem2)
        cp_pe.wait()
        cp_x0.wait()
        cp_x1.wait()

        def g_step(g, carry):
            base = g * 16
            row = base + iota
            for d in range(_DW):
                col = jnp.full((16,), d, jnp.int32)
                vals = plsc.load_gather(pe_v, [row, col])
                plsc.addupdate(x0_v.at[d, pl.ds(base, 16)], vals)
                plsc.addupdate(x1_v.at[d, pl.ds(base, 16)], vals)
            return carry

        lax.fori_loop(0, _NG, g_step, 0)

        pltpu.sync_copy(x0_v, o_hbm.at[0, pl.ds(d0, _DW), pl.ds(s0, _SC)])
        pltpu.sync_copy(x1_v, o_hbm.at[1, pl.ds(d0, _DW), pl.ds(s0, _SC)])


def kernel(x, pe_table):
    mesh = plsc.VectorSubcoreMesh(core_axis_name="c", subcore_axis_name="s")
    k = functools.partial(
        pl.kernel,
        mesh=mesh,
        compiler_params=pltpu.CompilerParams(needs_layout_passes=False),
        out_type=jax.ShapeDtypeStruct((_B, _D, _S), jnp.float32),
        scratch_types=[
            pltpu.VMEM((_SC, _DW), jnp.float32),
            pltpu.VMEM((_DW, _SC), jnp.float32),
            pltpu.VMEM((_DW, _SC), jnp.float32),
            pltpu.SemaphoreType.DMA,
            pltpu.SemaphoreType.DMA,
            pltpu.SemaphoreType.DMA,
        ],
    )(_sc_body)
    return k(x, pe_table)


# speedup vs baseline: 2.0109x; 2.0109x over previous
import functools

import jax
import jax.numpy as jnp
from jax import lax
from jax.experimental import pallas as pl
from jax.experimental.pallas import tpu as pltpu
from jax.experimental.pallas import tpu_sc as plsc

_B = 2
_D = 1024
_S = 8192
_NWD = 8
_NWS = 4
_DW = _D // _NWD
_SW = _S // _NWS
_SC = 128
_NCHUNK = _SW // _SC
_NG = _SC // 16


def _sc_body(x_hbm, pe_hbm, o_hbm, pe_v, x0_v, x1_v, sem_in, sem_out):
    wid = lax.axis_index("s") * 2 + lax.axis_index("c")
    d0 = (wid % _NWD) * _DW
    s_base = (wid // _NWD) * _SW
    iota = lax.iota(jnp.int32, 16)

    def fill(c, slot):
        s0 = s_base + c * _SC
        return (
            pltpu.async_copy(pe_hbm.at[pl.ds(s0, _SC), pl.ds(d0, _DW)],
                             pe_v.at[slot], sem_in[slot]),
            pltpu.async_copy(x_hbm.at[0, pl.ds(d0, _DW), pl.ds(s0, _SC)],
                             x0_v.at[slot], sem_in[slot]),
            pltpu.async_copy(x_hbm.at[1, pl.ds(d0, _DW), pl.ds(s0, _SC)],
                             x1_v.at[slot], sem_in[slot]),
        )

    def drain(c, slot):
        s0 = s_base + c * _SC
        return (
            pltpu.async_copy(x0_v.at[slot],
                             o_hbm.at[0, pl.ds(d0, _DW), pl.ds(s0, _SC)],
                             sem_out[slot]),
            pltpu.async_copy(x1_v.at[slot],
                             o_hbm.at[1, pl.ds(d0, _DW), pl.ds(s0, _SC)],
                             sem_out[slot]),
        )

    pending_fill = {0: fill(0, 0)}
    pending_out = {}

    for c in range(_NCHUNK):
        slot = c % 2
        if c + 1 < _NCHUNK:
            if (c - 1) in pending_out:
                for cp in pending_out.pop(c - 1):
                    cp.wait()
            pending_fill[c + 1] = fill(c + 1, 1 - slot)
        for cp in pending_fill.pop(c):
            cp.wait()

        @plsc.parallel_loop(0, _NG * _DW, unroll=8)
        def _transpose_add(i):
            g = i // _DW
            d = i % _DW
            base = g * 16
            row = base + iota
            col = jnp.full((16,), d, jnp.int32)
            vals = plsc.load_gather(pe_v.at[slot], [row, col])
            plsc.addupdate(x0_v.at[slot, d, pl.ds(base, 16)], vals)
            plsc.addupdate(x1_v.at[slot, d, pl.ds(base, 16)], vals)

        pending_out[c] = drain(c, slot)

    for c in sorted(pending_out):
        for cp in pending_out.pop(c):
            cp.wait()


def kernel(x, pe_table):
    mesh = plsc.VectorSubcoreMesh(core_axis_name="c", subcore_axis_name="s")
    k = functools.partial(
        pl.kernel,
        mesh=mesh,
        compiler_params=pltpu.CompilerParams(needs_layout_passes=False),
        out_type=jax.ShapeDtypeStruct((_B, _D, _S), jnp.float32),
        scratch_types=[
            pltpu.VMEM((2, _SC, _DW), jnp.float32),
            pltpu.VMEM((2, _DW, _SC), jnp.float32),
            pltpu.VMEM((2, _DW, _SC), jnp.float32),
            [pltpu.SemaphoreType.DMA, pltpu.SemaphoreType.DMA],
            [pltpu.SemaphoreType.DMA, pltpu.SemaphoreType.DMA],
        ],
    )(_sc_body)
    return k(x, pe_table)
